# baseline (speedup 1.0000x reference)
import functools

import jax
import jax.numpy as jnp
from jax import lax
from jax.experimental import pallas as pl
from jax.experimental.pallas import tpu as pltpu
from jax.experimental.pallas import tpu_sc as plsc

_B = 16384
_D = 64
_V = 1000000
_NC = 2
_NS = 16
_NW = _NC * _NS
_L = 16
_NBLK = (_V + 127) // 128
_BPR = _NBLK // _NW
_NEXTRA = _NBLK - _BPR * _NW
_W = 3
_NWIN = (_BPR + 1 + _W - 1) // _W + 1
_NWIN += _NWIN % 2
_MAXSTART = _NBLK - _W
_DUMP = _B


@functools.cache
def _gather_kernel():
    mesh = plsc.VectorSubcoreMesh(
        core_axis_name="c", subcore_axis_name="s",
        num_cores=_NC, num_subcores=_NS,
    )

    @functools.partial(
        pl.kernel,
        out_type=jax.ShapeDtypeStruct((_B + 1, 128), jnp.float32),
        mesh=mesh,
        scratch_types=[
            pltpu.VMEM((_B + _L,), jnp.int32),
            pltpu.VMEM((_B + _L,), jnp.int32),
            pltpu.VMEM((_B + _L,), jnp.int32),
            pltpu.VMEM((_B + _L,), jnp.int32),
            pltpu.VMEM((_D, _W * 128), jnp.float32),
            pltpu.VMEM((_D, _W * 128), jnp.float32),
            pltpu.VMEM((_L, 128), jnp.float32),
            pltpu.SemaphoreType.DMA,
            pltpu.SemaphoreType.DMA,
            pltpu.SemaphoreType.DMA,
            pltpu.SemaphoreType.DMA,
        ],
        compiler_params=pltpu.CompilerParams(needs_layout_passes=False),
    )
    def body(idx_hbm, tableT_hbm, out_hbm, lab_a, mylab, mypos, wpos,
             buf_a, buf_b, stag, sem_in, sem_a, sem_b, sem_s):
        wid = lax.axis_index("s") * _NC + lax.axis_index("c")
        lo = _BPR * wid + jnp.minimum(wid, _NEXTRA)
        lane = lax.iota(jnp.int32, _L)
        zeros = jnp.zeros((_L,), jnp.int32)
        lane_lo = lo * 128
        lane_hi = lane_lo + (_BPR + jnp.where(wid < _NEXTRA, 1, 0)) * 128

        pltpu.async_copy(idx_hbm, lab_a.at[pl.ds(0, _B)], sem_in).wait()

        def compress(g, n):
            lab = lab_a[pl.ds(g * _L, _L)]
            m = jnp.logical_and(lab >= lane_lo, lab < lane_hi)
            plsc.store_compressed(mylab.at[pl.ds(n, _L)], lab, mask=m)
            plsc.store_compressed(
                mypos.at[pl.ds(n, _L)], g * _L + lane, mask=m
            )
            return n + plsc.all_reduce_population_count(m)[0]

        n = lax.fori_loop(0, _B // _L, compress, 0, unroll=False)
        ng = lax.shift_right_logical(n + _L - 1, 4)

        def win_start(w):
            return jnp.minimum(lo + _W * w, _MAXSTART) * 128

        def fire(w, buf, sem):
            pltpu.async_copy(
                tableT_hbm.at[:, pl.ds(pl.multiple_of(win_start(w), 128),
                                       _W * 128)],
                buf, sem,
            )

        def drain(buf, sem):
            pltpu.make_async_copy(
                tableT_hbm.at[:, pl.ds(0, _W * 128)], buf, sem
            ).wait()

        def process(w, buf):
            start = win_start(w)

            def scan(g, mw):
                lab = lab16 = mylab[pl.ds(g * _L, _L)]
                valid = (g * _L + lane) < n
                m = jnp.logical_and(
                    valid,
                    jnp.logical_and(lab >= start, lab < start + _W * 128),
                )
                cnt = plsc.all_reduce_population_count(m)[0]

                @pl.when(cnt > 0)
                def _():
                    plsc.store_compressed(
                        lab_a.at[pl.ds(mw, _L)], lab16, mask=m
                    )
                    pos16 = mypos[pl.ds(g * _L, _L)]
                    plsc.store_compressed(
                        wpos.at[pl.ds(mw, _L)], pos16, mask=m
                    )

                return mw + cnt

            mw = lax.fori_loop(0, ng, scan, 0, unroll=False)

            def batch(b, _):
                rem = mw - b * _L
                wl = lab_a[pl.ds(b * _L, _L)]
                wp = mypos_pad = wpos[pl.ds(b * _L, _L)]
                wp = jnp.where(lane < rem, wp, _DUMP)
                for e in range(_L):
                    @pl.when(e < rem)
                    def _():
                        l_in = zeros + (wl[e] - start)
                        for c in range(_D // _L):
                            stag[e, pl.ds(c * _L, _L)] = plsc.load_gather(
                                buf, [c * _L + lane, l_in]
                            )
                pltpu.async_copy(stag, out_hbm.at[wp], sem_s).wait()
                return ()

            lax.fori_loop(0, lax.shift_right_logical(mw + _L - 1, 4),
                          batch, (), unroll=False)

        fire(0, buf_a, sem_a)

        def do_pair(p, _):
            fire(2 * p + 1, buf_b, sem_b)
            drain(buf_a, sem_a)
            process(2 * p, buf_a)

            @pl.when(p < _NWIN // 2 - 1)
            def _():
                fire(2 * p + 2, buf_a, sem_a)

            drain(buf_b, sem_b)
            process(2 * p + 1, buf_b)
            return ()

        lax.fori_loop(0, _NWIN // 2, do_pair, ())

    return body


def kernel(class_label, table, uncond_table):
    del uncond_table
    idx = class_label.astype(jnp.int32)
    out = _gather_kernel()(idx, table.T)
    return out[:_B, :_D].reshape(_B, 1, _D)


# speedup vs baseline: 6.1112x; 6.1112x over previous
import functools

import jax
import jax.numpy as jnp
from jax import lax
from jax.experimental import pallas as pl
from jax.experimental.pallas import tpu as pltpu
from jax.experimental.pallas import tpu_sc as plsc

_B = 16384
_D = 64
_NC = 2
_NS = 16
_NW = _NC * _NS
_BPW = _B // _NW
_C = 2
_NSET = 4
_NGRP = _BPW // 16
_L = 16


@functools.cache
def _gather_kernel():
    mesh = plsc.VectorSubcoreMesh(
        core_axis_name="c", subcore_axis_name="s",
        num_cores=_NC, num_subcores=_NS,
    )

    block_types = [
        pltpu.VMEM((_D, 128), jnp.float32) for _ in range(_NSET * _C)
    ]

    @functools.partial(
        pl.kernel,
        out_type=jax.ShapeDtypeStruct((_D, _B), jnp.float32),
        mesh=mesh,
        scratch_types=[
            pltpu.VMEM((_BPW,), jnp.int32),
            *block_types,
            pltpu.VMEM((_D, _BPW), jnp.float32),
            pltpu.SemaphoreType.DMA,
            *[pltpu.SemaphoreType.DMA for _ in range(_NSET)],
        ],
        compiler_params=pltpu.CompilerParams(needs_layout_passes=False),
    )
    def body(idx_hbm, tableT_hbm, outT_hbm, lab_v, *rest):
        bufs = [
            rest[s * _C:(s + 1) * _C] for s in range(_NSET)
        ]
        outT_v = rest[_NSET * _C]
        sem_in = rest[_NSET * _C + 1]
        sems = rest[_NSET * _C + 2:]
        wid = lax.axis_index("s") * _NC + lax.axis_index("c")
        base = wid * _BPW
        pltpu.async_copy(idx_hbm.at[pl.ds(base, _BPW)], lab_v, sem_in).wait()

        lane = lax.iota(jnp.int32, _L)
        zeros = jnp.zeros((_L,), jnp.int32)
        nchunk = _L // _C

        def fire(lab16, lbase, s):
            for e in range(_C):
                blk0 = pl.multiple_of(
                    lax.bitwise_and(lab16[lbase + e], -128), 128
                )
                pltpu.async_copy(
                    tableT_hbm.at[:, pl.ds(blk0, 128)], bufs[s][e], sems[s]
                )

        def drain(s):
            for e in range(_C):
                pltpu.make_async_copy(
                    tableT_hbm.at[:, pl.ds(0, 128)], bufs[s][e], sems[s]
                ).wait()

        def extract(lab16, lbase, off, s):
            for e in range(_C):
                l_vec = zeros + lax.bitwise_and(lab16[lbase + e], 127)
                p_vec = zeros + (off + e)
                for c in range(_D // _L):
                    val = plsc.load_gather(bufs[s][e], [c * _L + lane, l_vec])
                    plsc.store_scatter(outT_v, [c * _L + lane, p_vec], val)

        lab0 = lab_v[pl.ds(0, _L)]
        for j in range(_NSET - 1):
            fire(lab0, j * _C, j)

        def do_group(g, _):
            lab16 = lab_v[pl.ds(g * _L, _L)]
            off = g * _L
            for j in range(nchunk):
                s = j % _NSET
                fj = j + _NSET - 1
                if fj < nchunk:
                    fire(lab16, fj * _C, (fj % _NSET))
                else:
                    fj -= nchunk

                    @pl.when(g < _NGRP - 1)
                    def _(fj=fj):
                        lab_n = lab_v[pl.ds((g + 1) * _L, _L)]
                        fire(lab_n, fj * _C, fj % _NSET)

                drain(s)
                extract(lab16, j * _C, off + j * _C, s)
            return ()

        lax.fori_loop(0, _NGRP, do_group, ())
        pltpu.sync_copy(outT_v, outT_hbm.at[:, pl.ds(base, _BPW)])

    return body


def kernel(class_label, table, uncond_table):
    del uncond_table
    idx = class_label.astype(jnp.int32)
    outT = _gather_kernel()(idx, table.T)
    return outT.T.reshape(_B, 1, _D)


# speedup vs baseline: 6.5687x; 1.0749x over previous
import functools

import jax
import jax.numpy as jnp
from jax import lax
from jax.experimental import pallas as pl
from jax.experimental.pallas import tpu as pltpu
from jax.experimental.pallas import tpu_sc as plsc

_B = 16384
_D = 64
_NC = 2
_NS = 16
_NW = _NC * _NS
_BPW = _B // _NW
_C = 1
_NSET = 8
_NGRP = _BPW // 16
_L = 16


@functools.cache
def _gather_kernel():
    mesh = plsc.VectorSubcoreMesh(
        core_axis_name="c", subcore_axis_name="s",
        num_cores=_NC, num_subcores=_NS,
    )

    block_types = [
        pltpu.VMEM((_D, 128), jnp.float32) for _ in range(_NSET * _C)
    ]

    @functools.partial(
        pl.kernel,
        out_type=jax.ShapeDtypeStruct((_D, _B), jnp.float32),
        mesh=mesh,
        scratch_types=[
            pltpu.VMEM((_BPW,), jnp.int32),
            *block_types,
            pltpu.VMEM((_D, _BPW), jnp.float32),
            pltpu.SemaphoreType.DMA,
            *[pltpu.SemaphoreType.DMA for _ in range(_NSET)],
        ],
        compiler_params=pltpu.CompilerParams(needs_layout_passes=False),
    )
    def body(idx_hbm, tableT_hbm, outT_hbm, lab_v, *rest):
        bufs = [
            rest[s * _C:(s + 1) * _C] for s in range(_NSET)
        ]
        outT_v = rest[_NSET * _C]
        sem_in = rest[_NSET * _C + 1]
        sems = rest[_NSET * _C + 2:]
        wid = lax.axis_index("s") * _NC + lax.axis_index("c")
        base = wid * _BPW
        pltpu.async_copy(idx_hbm.at[pl.ds(base, _BPW)], lab_v, sem_in).wait()

        lane = lax.iota(jnp.int32, _L)
        zeros = jnp.zeros((_L,), jnp.int32)
        nchunk = _L // _C

        def fire(lab16, lbase, s):
            for e in range(_C):
                blk0 = pl.multiple_of(
                    lax.bitwise_and(lab16[lbase + e], -128), 128
                )
                pltpu.async_copy(
                    tableT_hbm.at[:, pl.ds(blk0, 128)], bufs[s][e], sems[s]
                )

        def drain(s):
            for e in range(_C):
                pltpu.make_async_copy(
                    tableT_hbm.at[:, pl.ds(0, 128)], bufs[s][e], sems[s]
                ).wait()

        def extract(lab16, lbase, off, s):
            for e in range(_C):
                l_vec = zeros + lax.bitwise_and(lab16[lbase + e], 127)
                p_vec = zeros + (off + e)
                for c in range(_D // _L):
                    val = plsc.load_gather(bufs[s][e], [c * _L + lane, l_vec])
                    plsc.store_scatter(outT_v, [c * _L + lane, p_vec], val)

        lab0 = lab_v[pl.ds(0, _L)]
        for j in range(_NSET - 1):
            fire(lab0, j * _C, j)

        def do_group(g, _):
            lab16 = lab_v[pl.ds(g * _L, _L)]
            off = g * _L
            for j in range(nchunk):
                s = j % _NSET
                fj = j + _NSET - 1
                if fj < nchunk:
                    fire(lab16, fj * _C, (fj % _NSET))
                else:
                    fj -= nchunk

                    @pl.when(g < _NGRP - 1)
                    def _(fj=fj):
                        lab_n = lab_v[pl.ds((g + 1) * _L, _L)]
                        fire(lab_n, fj * _C, fj % _NSET)

                drain(s)
                extract(lab16, j * _C, off + j * _C, s)
            return ()

        lax.fori_loop(0, _NGRP, do_group, ())
        pltpu.sync_copy(outT_v, outT_hbm.at[:, pl.ds(base, _BPW)])

    return body


def kernel(class_label, table, uncond_table):
    del uncond_table
    idx = class_label.astype(jnp.int32)
    outT = _gather_kernel()(idx, table.T)
    return outT.T.reshape(_B, 1, _D)
